# baseline (speedup 1.0000x reference)
import jax
import jax.numpy as jnp
from jax import lax
from jax.experimental import pallas as pl
from jax.experimental.pallas import tpu as pltpu
from jax.experimental.pallas import tpu_sc as plsc

N = 10000
NP = 10240
E = 320000
D = 128
G = 16
NPG = N // G

NC = 2
NS = 16
NW = NC * NS
K = 80
NCHUNK = 128
EPWP = NCHUNK * K
NBR = 1
NBI = 2
RPT = NP // NS
DUMP = N


def _sc_agg_body(h_hbm, pairs_hbm, zrows_hbm, out_hbm,
                 agg_sh, pairs_v, rows_v, isems, gsems, ssems):
    cid = lax.axis_index("c")
    sid = lax.axis_index("s")
    wid = sid * NC + cid

    for q in range(NBI):
        pltpu.async_copy(pairs_hbm.at[wid, q], pairs_v.at[q], isems.at[q])
    pltpu.sync_copy(zrows_hbm, agg_sh.at[pl.ds(sid * RPT, RPT)])
    plsc.subcore_barrier()

    def body(j, _):
        for q in range(NBI):
            i = j * NBI + q
            pltpu.make_async_copy(pairs_hbm.at[wid, i], pairs_v.at[q],
                                  isems.at[q]).wait()
            cp = pltpu.async_copy(h_hbm.at[pairs_v.at[q, 0]], rows_v.at[0],
                                  gsems.at[0])
            cp.wait()
            pltpu.sync_copy(rows_v.at[0], agg_sh.at[pairs_v.at[q, 1]],
                            add=True)

            @pl.when(i + NBI < NCHUNK)
            def _():
                pltpu.async_copy(pairs_hbm.at[wid, i + NBI], pairs_v.at[q],
                                 isems.at[q])
        return 0

    lax.fori_loop(0, NCHUNK // NBI, body, 0)
    plsc.subcore_barrier()
    pltpu.sync_copy(agg_sh.at[pl.ds(sid * RPT, RPT)],
                    out_hbm.at[cid, pl.ds(sid * RPT, RPT)])


_sc_agg = pl.kernel(
    _sc_agg_body,
    out_type=jax.ShapeDtypeStruct((NC, NP, D), jnp.float32),
    mesh=plsc.VectorSubcoreMesh(core_axis_name="c", subcore_axis_name="s"),
    scratch_types=[
        pltpu.VMEM_SHARED((NP, D), jnp.float32),
        pltpu.VMEM((NBI, 2, K), jnp.int32),
        pltpu.VMEM((NBR, K, D), jnp.float32),
        pltpu.SemaphoreType.DMA((NBI,)),
        pltpu.SemaphoreType.DMA((NBR,)),
        pltpu.SemaphoreType.DMA((NBR,)),
    ],
)


def _tc_layer_body(p_ref, w_ref, b_ref, o_ref):
    a = p_ref[0] + p_ref[1]
    z = jnp.dot(a, w_ref[...], preferred_element_type=jnp.float32) + b_ref[...]
    o_ref[...] = jnp.maximum(z, 0.2 * z)


def _tc_layer(partials, w, b):
    R = 2048
    return pl.pallas_call(
        _tc_layer_body,
        out_shape=jax.ShapeDtypeStruct((NP, D), jnp.float32),
        grid=(NP // R,),
        in_specs=[
            pl.BlockSpec((NC, R, D), lambda i: (0, i, 0)),
            pl.BlockSpec((D, D), lambda i: (0, 0)),
            pl.BlockSpec((1, D), lambda i: (0, 0)),
        ],
        out_specs=pl.BlockSpec((R, D), lambda i: (i, 0)),
    )(partials, w, b.reshape(1, D))


def _tc_pool_body(p_ref, w_ref, b_ref, o_ref):
    a = p_ref[0] + p_ref[1]
    z = jnp.dot(a, w_ref[...], preferred_element_type=jnp.float32) + b_ref[...]
    h = jnp.maximum(z, 0.2 * z)
    hh = h.reshape(-1, NPG, D)
    o_ref[...] = jnp.sum(hh, axis=1) * (1.0 / NPG)


def _tc_pool(partials, w, b):
    GB = 8
    R = GB * NPG
    return pl.pallas_call(
        _tc_pool_body,
        out_shape=jax.ShapeDtypeStruct((G, D), jnp.float32),
        grid=(G // GB,),
        in_specs=[
            pl.BlockSpec((NC, R, D), lambda i: (0, i, 0)),
            pl.BlockSpec((D, D), lambda i: (0, 0)),
            pl.BlockSpec((1, D), lambda i: (0, 0)),
        ],
        out_specs=pl.BlockSpec((GB, D), lambda i: (i, 0)),
    )(partials, w, b.reshape(1, D))


def _tc_mlp_body(p_ref, w1_ref, b1_ref, w2_ref, b2_ref, o_ref):
    z = jnp.dot(p_ref[...], w1_ref[...], preferred_element_type=jnp.float32)
    z = z + b1_ref[...]
    g = jnp.maximum(z, 0.2 * z)
    o_ref[...] = jnp.dot(g, w2_ref[...],
                         preferred_element_type=jnp.float32) + b2_ref[...]


def _tc_mlp(pooled, w1, b1, w2, b2):
    C = w2.shape[1]
    H2 = w1.shape[1]
    return pl.pallas_call(
        _tc_mlp_body,
        out_shape=jax.ShapeDtypeStruct((G, C), jnp.float32),
    )(pooled, w1, b1.reshape(1, H2), w2, b2.reshape(1, C))


def kernel(x, edge_index, batch, W1, b1, W2, b2, lin1_w, lin1_b, lin2_w, lin2_b):
    epw = E // NW
    pad = EPWP - epw
    srcp = jnp.concatenate(
        [edge_index[0].reshape(NW, epw),
         jnp.zeros((NW, pad), jnp.int32)], axis=1).reshape(NW, NCHUNK, K)
    dstp = jnp.concatenate(
        [edge_index[1].reshape(NW, epw),
         jnp.full((NW, pad), DUMP, jnp.int32)], axis=1).reshape(NW, NCHUNK, K)
    pairs = jnp.stack([srcp, dstp], axis=2)
    zrows = jnp.zeros((RPT, D), jnp.float32)

    p1 = _sc_agg(x, pairs, zrows)
    h1 = _tc_layer(p1, W1, b1)
    p2 = _sc_agg(h1, pairs, zrows)
    pooled = _tc_pool(p2, W2 + W2, b2)
    return _tc_mlp(pooled, lin1_w, lin1_b, lin2_w, lin2_b)


# speedup vs baseline: 2.2266x; 2.2266x over previous
import jax
import jax.numpy as jnp
from jax import lax
from jax.experimental import pallas as pl
from jax.experimental.pallas import tpu as pltpu
from jax.experimental.pallas import tpu_sc as plsc

N = 10000
NP = 10240
E = 320000
D = 128
G = 16
NPG = N // G

NC = 2
NS = 16
NW = NC * NS
K = 80
NCHUNK = 125
RPT = NP // NS


def _sc_agg_body(h_hbm, src_hbm, dst_hbm, zrows_hbm, out_hbm,
                 agg_sh, s0_v, s1_v, d0_v, d1_v, rows_v, isems, gsems):
    cid = lax.axis_index("c")
    sid = lax.axis_index("s")
    wid = sid * NC + cid
    base = wid * NCHUNK * K
    svs = (s0_v, s1_v)
    dvs = (d0_v, d1_v)

    for q in range(2):
        pltpu.async_copy(src_hbm.at[pl.ds(base + q * K, K)], svs[q],
                         isems.at[q])
        pltpu.async_copy(dst_hbm.at[pl.ds(base + q * K, K)], dvs[q],
                         isems.at[q])
    pltpu.sync_copy(zrows_hbm, agg_sh.at[pl.ds(sid * RPT, RPT)])
    plsc.subcore_barrier()

    def chunk_step(i, q):
        off = base + i * K
        pltpu.make_async_copy(src_hbm.at[pl.ds(off, K)], svs[q],
                              isems.at[q]).wait()
        pltpu.make_async_copy(dst_hbm.at[pl.ds(off, K)], dvs[q],
                              isems.at[q]).wait()
        pltpu.async_copy(h_hbm.at[svs[q]], rows_v, gsems.at[0]).wait()
        pltpu.sync_copy(rows_v, agg_sh.at[dvs[q]], add=True)

        @pl.when(i + 2 < NCHUNK)
        def _():
            pltpu.async_copy(src_hbm.at[pl.ds(off + 2 * K, K)], svs[q],
                             isems.at[q])
            pltpu.async_copy(dst_hbm.at[pl.ds(off + 2 * K, K)], dvs[q],
                             isems.at[q])

    def body(j, _):
        chunk_step(2 * j, 0)
        chunk_step(2 * j + 1, 1)
        return 0

    lax.fori_loop(0, NCHUNK // 2, body, 0)
    for i in range((NCHUNK // 2) * 2, NCHUNK):
        chunk_step(i, i % 2)
    plsc.subcore_barrier()
    pltpu.sync_copy(agg_sh.at[pl.ds(sid * RPT, RPT)],
                    out_hbm.at[cid, pl.ds(sid * RPT, RPT)])


_sc_agg = pl.kernel(
    _sc_agg_body,
    out_type=jax.ShapeDtypeStruct((NC, NP, D), jnp.float32),
    mesh=plsc.VectorSubcoreMesh(core_axis_name="c", subcore_axis_name="s"),
    scratch_types=[
        pltpu.VMEM_SHARED((NP, D), jnp.float32),
        pltpu.VMEM((K,), jnp.int32),
        pltpu.VMEM((K,), jnp.int32),
        pltpu.VMEM((K,), jnp.int32),
        pltpu.VMEM((K,), jnp.int32),
        pltpu.VMEM((K, D), jnp.float32),
        pltpu.SemaphoreType.DMA((2,)),
        pltpu.SemaphoreType.DMA((1,)),
    ],
)


def _tc_layer_body(p_ref, w_ref, b_ref, o_ref):
    a = p_ref[0] + p_ref[1]
    z = jnp.dot(a, w_ref[...], preferred_element_type=jnp.float32) + b_ref[...]
    o_ref[...] = jnp.maximum(z, 0.2 * z)


def _tc_layer(partials, w, b):
    R = 2048
    return pl.pallas_call(
        _tc_layer_body,
        out_shape=jax.ShapeDtypeStruct((NP, D), jnp.float32),
        grid=(NP // R,),
        in_specs=[
            pl.BlockSpec((NC, R, D), lambda i: (0, i, 0)),
            pl.BlockSpec((D, D), lambda i: (0, 0)),
            pl.BlockSpec((1, D), lambda i: (0, 0)),
        ],
        out_specs=pl.BlockSpec((R, D), lambda i: (i, 0)),
    )(partials, w, b.reshape(1, D))


def _tc_pool_body(p_ref, w_ref, b_ref, o_ref):
    a = p_ref[0] + p_ref[1]
    z = jnp.dot(a, w_ref[...], preferred_element_type=jnp.float32) + b_ref[...]
    h = jnp.maximum(z, 0.2 * z)
    hh = h.reshape(-1, NPG, D)
    o_ref[...] = jnp.sum(hh, axis=1) * (1.0 / NPG)


def _tc_pool(partials, w, b):
    GB = 8
    R = GB * NPG
    return pl.pallas_call(
        _tc_pool_body,
        out_shape=jax.ShapeDtypeStruct((G, D), jnp.float32),
        grid=(G // GB,),
        in_specs=[
            pl.BlockSpec((NC, R, D), lambda i: (0, i, 0)),
            pl.BlockSpec((D, D), lambda i: (0, 0)),
            pl.BlockSpec((1, D), lambda i: (0, 0)),
        ],
        out_specs=pl.BlockSpec((GB, D), lambda i: (i, 0)),
    )(partials, w, b.reshape(1, D))


def _tc_mlp_body(p_ref, w1_ref, b1_ref, w2_ref, b2_ref, o_ref):
    z = jnp.dot(p_ref[...], w1_ref[...], preferred_element_type=jnp.float32)
    z = z + b1_ref[...]
    g = jnp.maximum(z, 0.2 * z)
    o_ref[...] = jnp.dot(g, w2_ref[...],
                         preferred_element_type=jnp.float32) + b2_ref[...]


def _tc_mlp(pooled, w1, b1, w2, b2):
    C = w2.shape[1]
    H2 = w1.shape[1]
    return pl.pallas_call(
        _tc_mlp_body,
        out_shape=jax.ShapeDtypeStruct((G, C), jnp.float32),
    )(pooled, w1, b1.reshape(1, H2), w2, b2.reshape(1, C))


def kernel(x, edge_index, batch, W1, b1, W2, b2, lin1_w, lin1_b, lin2_w, lin2_b):
    src = edge_index[0]
    dst = edge_index[1]
    zrows = jnp.zeros((RPT, D), jnp.float32)

    p1 = _sc_agg(x, src, dst, zrows)
    h1 = _tc_layer(p1, W1, b1)
    p2 = _sc_agg(h1, src, dst, zrows)
    pooled = _tc_pool(p2, W2 + W2, b2)
    return _tc_mlp(pooled, lin1_w, lin1_b, lin2_w, lin2_b)


# speedup vs baseline: 3.6150x; 1.6236x over previous
import jax
import jax.numpy as jnp
from jax import lax
from jax.experimental import pallas as pl
from jax.experimental.pallas import tpu as pltpu
from jax.experimental.pallas import tpu_sc as plsc

N = 10000
NP = 10240
E = 320000
D = 128
G = 16
NPG = N // G

NC = 2
NS = 16
NW = NC * NS
K = 80
NCHUNK = 125
RPT = NP // NS


def _sc_agg_body(h_hbm, src_hbm, dst_hbm, zrows_hbm, out_hbm,
                 agg_sh, s0_v, s1_v, s2_v, s3_v, d0_v, d1_v, d2_v, d3_v,
                 r0_v, r1_v, isems, gsems):
    cid = lax.axis_index("c")
    sid = lax.axis_index("s")
    wid = sid * NC + cid
    base = wid * NCHUNK * K
    svs = (s0_v, s1_v, s2_v, s3_v)
    dvs = (d0_v, d1_v, d2_v, d3_v)
    rvs = (r0_v, r1_v)

    for q in range(4):
        pltpu.async_copy(src_hbm.at[pl.ds(base + q * K, K)], svs[q],
                         isems.at[q])
        pltpu.async_copy(dst_hbm.at[pl.ds(base + q * K, K)], dvs[q],
                         isems.at[q])
    pltpu.sync_copy(zrows_hbm, agg_sh.at[pl.ds(sid * RPT, RPT)])
    for q in range(2):
        pltpu.make_async_copy(src_hbm.at[pl.ds(base + q * K, K)], svs[q],
                              isems.at[q]).wait()
        pltpu.make_async_copy(dst_hbm.at[pl.ds(base + q * K, K)], dvs[q],
                              isems.at[q]).wait()
        pltpu.async_copy(h_hbm.at[svs[q]], rvs[q], gsems.at[q])
    plsc.subcore_barrier()

    def chunk_step(i, q4, q2):
        off = base + i * K
        pltpu.make_async_copy(h_hbm.at[svs[q4]], rvs[q2],
                              gsems.at[q2]).wait()
        pltpu.sync_copy(rvs[q2], agg_sh.at[dvs[q4]], add=True)

        @pl.when(i + 4 < NCHUNK)
        def _():
            pltpu.async_copy(src_hbm.at[pl.ds(off + 4 * K, K)], svs[q4],
                             isems.at[q4])
            pltpu.async_copy(dst_hbm.at[pl.ds(off + 4 * K, K)], dvs[q4],
                             isems.at[q4])

        @pl.when(i + 2 < NCHUNK)
        def _():
            qn = (q4 + 2) % 4
            pltpu.make_async_copy(src_hbm.at[pl.ds(off + 2 * K, K)],
                                  svs[qn], isems.at[qn]).wait()
            pltpu.make_async_copy(dst_hbm.at[pl.ds(off + 2 * K, K)],
                                  dvs[qn], isems.at[qn]).wait()
            pltpu.async_copy(h_hbm.at[svs[qn]], rvs[q2], gsems.at[q2])

    def body(j, _):
        for u in range(4):
            chunk_step(4 * j + u, u, u % 2)
        return 0

    lax.fori_loop(0, NCHUNK // 4, body, 0)
    for i in range((NCHUNK // 4) * 4, NCHUNK):
        chunk_step(i, i % 4, i % 2)
    plsc.subcore_barrier()
    pltpu.sync_copy(agg_sh.at[pl.ds(sid * RPT, RPT)],
                    out_hbm.at[cid, pl.ds(sid * RPT, RPT)])


_sc_agg = pl.kernel(
    _sc_agg_body,
    out_type=jax.ShapeDtypeStruct((NC, NP, D), jnp.float32),
    mesh=plsc.VectorSubcoreMesh(core_axis_name="c", subcore_axis_name="s"),
    scratch_types=(
        [pltpu.VMEM_SHARED((NP, D), jnp.float32)]
        + [pltpu.VMEM((K,), jnp.int32) for _ in range(8)]
        + [pltpu.VMEM((K, D), jnp.float32) for _ in range(2)]
        + [pltpu.SemaphoreType.DMA((4,)), pltpu.SemaphoreType.DMA((2,))]
    ),
)


def _tc_layer_body(p_ref, w_ref, b_ref, o_ref):
    a = p_ref[0] + p_ref[1]
    z = jnp.dot(a, w_ref[...], preferred_element_type=jnp.float32) + b_ref[...]
    o_ref[...] = jnp.maximum(z, 0.2 * z)


def _tc_layer(partials, w, b):
    R = 2048
    return pl.pallas_call(
        _tc_layer_body,
        out_shape=jax.ShapeDtypeStruct((NP, D), jnp.float32),
        grid=(NP // R,),
        in_specs=[
            pl.BlockSpec((NC, R, D), lambda i: (0, i, 0)),
            pl.BlockSpec((D, D), lambda i: (0, 0)),
            pl.BlockSpec((1, D), lambda i: (0, 0)),
        ],
        out_specs=pl.BlockSpec((R, D), lambda i: (i, 0)),
    )(partials, w, b.reshape(1, D))


def _tc_pool_body(p_ref, w_ref, b_ref, o_ref):
    a = p_ref[0] + p_ref[1]
    z = jnp.dot(a, w_ref[...], preferred_element_type=jnp.float32) + b_ref[...]
    h = jnp.maximum(z, 0.2 * z)
    hh = h.reshape(-1, NPG, D)
    o_ref[...] = jnp.sum(hh, axis=1) * (1.0 / NPG)


def _tc_pool(partials, w, b):
    GB = 8
    R = GB * NPG
    return pl.pallas_call(
        _tc_pool_body,
        out_shape=jax.ShapeDtypeStruct((G, D), jnp.float32),
        grid=(G // GB,),
        in_specs=[
            pl.BlockSpec((NC, R, D), lambda i: (0, i, 0)),
            pl.BlockSpec((D, D), lambda i: (0, 0)),
            pl.BlockSpec((1, D), lambda i: (0, 0)),
        ],
        out_specs=pl.BlockSpec((GB, D), lambda i: (i, 0)),
    )(partials, w, b.reshape(1, D))


def _tc_mlp_body(p_ref, w1_ref, b1_ref, w2_ref, b2_ref, o_ref):
    z = jnp.dot(p_ref[...], w1_ref[...], preferred_element_type=jnp.float32)
    z = z + b1_ref[...]
    g = jnp.maximum(z, 0.2 * z)
    o_ref[...] = jnp.dot(g, w2_ref[...],
                         preferred_element_type=jnp.float32) + b2_ref[...]


def _tc_mlp(pooled, w1, b1, w2, b2):
    C = w2.shape[1]
    H2 = w1.shape[1]
    return pl.pallas_call(
        _tc_mlp_body,
        out_shape=jax.ShapeDtypeStruct((G, C), jnp.float32),
    )(pooled, w1, b1.reshape(1, H2), w2, b2.reshape(1, C))


def kernel(x, edge_index, batch, W1, b1, W2, b2, lin1_w, lin1_b, lin2_w, lin2_b):
    src = edge_index[0]
    dst = edge_index[1]
    zrows = jnp.zeros((RPT, D), jnp.float32)

    p1 = _sc_agg(x, src, dst, zrows)
    h1 = _tc_layer(p1, W1, b1)
    p2 = _sc_agg(h1, src, dst, zrows)
    pooled = _tc_pool(p2, W2 + W2, b2)
    return _tc_mlp(pooled, lin1_w, lin1_b, lin2_w, lin2_b)
